# Initial kernel scaffold; baseline (speedup 1.0000x reference)
import jax
import jax.numpy as jnp
from jax.experimental import pallas as pl


def kernel(mu, sigma, edge_index, edge_dist, edge_conf, edge_angle, edge_depth_diff, msg_W1, msg_b1, msg_W2, msg_b2, mu_W1, mu_b1, mu_W2, mu_b2, sig_W1, sig_b1, sig_W2, sig_b2):
    raise NotImplementedError("write your pallas kernel here")


# speedup vs baseline: 3.7410x; 3.7410x over previous
import functools

import jax
import jax.numpy as jnp
from jax import lax
from jax.experimental import pallas as pl
from jax.experimental.pallas import tpu as pltpu
from jax.experimental.pallas import tpu_sc as plsc

N = 1024
E = 16384
H = 128
NC = 2
NS = 16
NW = NC * NS
ROWS_W = N // NW
CELLS_W = ROWS_W * N
EW = E // NW
CHUNK = 256
NCHUNK = EW // CHUNK
ROWS_S = N // NS

_HI = jax.lax.Precision.HIGHEST


def _proj_body(mu_ref, sig_ref, ef4_ref, w1_ref, b1_ref, p_ref, efp_ref):
    w1a = w1_ref[0:H, :]
    w1b = w1_ref[H:2 * H, :]
    w1c = w1_ref[2 * H:, :]
    p_ref[...] = (jnp.dot(mu_ref[...], w1a, precision=_HI)
                  + jnp.dot(sig_ref[...], w1b, precision=_HI)
                  + b1_ref[...])
    efp_ref[...] = jnp.dot(ef4_ref[...], w1c, precision=_HI)


def _adj_body(flat_hbm, dval_hbm, fill_hbm, adj_hbm, idx_v, val_v, tile_v):
    c = lax.axis_index("c")
    s = lax.axis_index("s")
    wid = c * NS + s
    base = wid * CELLS_W
    pltpu.sync_copy(fill_hbm, tile_v)
    pltpu.sync_copy(flat_hbm, idx_v)
    pltpu.sync_copy(dval_hbm, val_v)

    def body(g, carry):
        idx = idx_v[pl.ds(g * 16, 16)]
        val = val_v[pl.ds(g * 16, 16)]
        loc = idx - base
        msk = (loc >= 0) & (loc < CELLS_W)
        locc = jnp.clip(loc, 0, CELLS_W - 1)
        plsc.store_scatter(tile_v, [locc], val, mask=msk)
        return carry

    lax.fori_loop(0, E // 16, body, 0)
    pltpu.sync_copy(tile_v, adj_hbm.at[pl.ds(base, CELLS_W)])


def _twohop_body(adj_ref, g_ref):
    a = adj_ref[...]
    m = (a >= 0.0).astype(jnp.float32)
    dm = jnp.maximum(a, 0.0)
    pc = jnp.dot(m, m, precision=_HI)
    s = (jnp.dot(dm, m, precision=_HI) + jnp.dot(m, dm, precision=_HI))
    g_ref[...] = jnp.where(pc > 0.0, s / jnp.maximum(pc, 1.0), -1.0)


def _edge_body(srcr_hbm, dstr_hbm, flatr_hbm, dr_hbm, g_hbm, pn_hbm, efr_hbm,
               agg_hbm, wsum_hbm, rsum_hbm, deg_hbm, resid_hbm,
               src_a, src_b, dst_a, dst_b, flat_a, flat_b, d_a, d_b,
               g_a, g_b, w_a, w_b, r_a, r_b, one_v, zv_v,
               prow_v, efrow_v,
               agg_s, wsum_s, rsum_s, deg_s, sem):
    src_c = (src_a, src_b)
    dst_c = (dst_a, dst_b)
    flat_c = (flat_a, flat_b)
    d_c = (d_a, d_b)
    g_c = (g_a, g_b)
    w_c = (w_a, w_b)
    r_c = (r_a, r_b)
    c = lax.axis_index("c")
    s = lax.axis_index("s")
    wid = c * NS + s

    zsl = pl.ds(s * ROWS_S, ROWS_S)
    zero16 = jnp.zeros((16,), _f32)

    def zrow_body(i, carry):
        for j in range(H // 16):
            efrow_v[i, pl.ds(j * 16, 16)] = zero16
        return carry

    lax.fori_loop(0, ROWS_S, zrow_body, 0)
    for g16 in range(ROWS_S // 16):
        zv_v[pl.ds(g16 * 16, 16)] = zero16
    pltpu.sync_copy(efrow_v.at[pl.ds(0, ROWS_S)], agg_s.at[zsl])
    pltpu.sync_copy(zv_v, wsum_s.at[zsl])
    pltpu.sync_copy(zv_v, rsum_s.at[zsl])
    pltpu.sync_copy(zv_v, deg_s.at[zsl])

    for ci in range(NCHUNK):
        pltpu.sync_copy(srcr_hbm.at[wid, ci], src_c[ci])
        pltpu.sync_copy(dstr_hbm.at[wid, ci], dst_c[ci])
        pltpu.sync_copy(flatr_hbm.at[wid, ci], flat_c[ci])
        pltpu.sync_copy(dr_hbm.at[wid, ci], d_c[ci])
    for g16 in range(16):
        one_v[pl.ds(g16 * 16, 16)] = jnp.full((16,), 1.0, jnp.float32)

    plsc.subcore_barrier()

    for ci in range(NCHUNK):
        pltpu.async_copy(g_hbm.at[flat_c[ci]], g_c[ci], sem).wait()
        for g16 in range(CHUNK // 16):
            sl = pl.ds(g16 * 16, 16)
            gg = g_c[ci][sl]
            dd = d_c[ci][sl]
            rr = jnp.where(gg >= 0.0, jnp.abs(dd - gg), 0.0)
            r_c[ci][sl] = rr
            w_c[ci][sl] = jnp.exp(-rr)
        pltpu.async_copy(pn_hbm.at[src_c[ci]], prow_v, sem).wait()
        pltpu.sync_copy(efr_hbm.at[wid, ci], efrow_v)
        w_ref = w_c[ci]

        def ebody(e, carry):
            wspl = plsc.load_gather(w_ref, [jnp.full((16,), e, jnp.int32)])
            for j in range(H // 16):
                sl = pl.ds(j * 16, 16)
                hv = jnp.maximum(prow_v[e, sl] + efrow_v[e, sl], 0.0) * wspl
                efrow_v[e, sl] = hv
            return carry

        lax.fori_loop(0, CHUNK, ebody, 0)

        pltpu.sync_copy(efrow_v, agg_s.at[dst_c[ci]], add=True)
        pltpu.sync_copy(w_c[ci], wsum_s.at[dst_c[ci]], add=True)
        pltpu.sync_copy(r_c[ci], rsum_s.at[dst_c[ci]], add=True)
        pltpu.sync_copy(one_v, deg_s.at[dst_c[ci]], add=True)

        pltpu.sync_copy(r_c[ci], resid_hbm.at[wid, ci])

    plsc.subcore_barrier()

    pltpu.sync_copy(agg_s.at[zsl], efrow_v.at[pl.ds(0, ROWS_S)])
    pltpu.sync_copy(efrow_v.at[pl.ds(0, ROWS_S)], agg_hbm.at[c, zsl])
    pltpu.sync_copy(wsum_s.at[zsl], zv_v)
    pltpu.sync_copy(zv_v, wsum_hbm.at[c, zsl])
    pltpu.sync_copy(rsum_s.at[zsl], zv_v)
    pltpu.sync_copy(zv_v, rsum_hbm.at[c, zsl])
    pltpu.sync_copy(deg_s.at[zsl], zv_v)
    pltpu.sync_copy(zv_v, deg_hbm.at[c, zsl])


def _final_body(agg_ref, ws_ref, rs_ref, dg_ref, mu_ref,
                w2_ref, b2_ref, muw1_ref, mub1_ref, muw2_ref, mub2_ref,
                sgw1_ref, sgb1_ref, sgw2_ref, sgb2_ref,
                munew_ref, signew_ref):
    a = agg_ref[0] + agg_ref[1]
    ws = ws_ref[0] + ws_ref[1]
    rs = rs_ref[0] + rs_ref[1]
    dg = dg_ref[0] + dg_ref[1]
    agg = ((jnp.dot(a, w2_ref[...], precision=_HI) + ws * b2_ref[...])
           / jnp.maximum(ws, 1e-8))
    hmu = jnp.maximum(jnp.dot(agg, muw1_ref[...], precision=_HI) + mub1_ref[...], 0.0)
    munew_ref[...] = mu_ref[...] + jnp.dot(hmu, muw2_ref[...], precision=_HI) + mub2_ref[...]
    rmean = rs / jnp.maximum(dg, 1.0)
    sgw1a = sgw1_ref[0:H, :]
    sgw1b = sgw1_ref[H:H + 1, :]
    hsg = jnp.maximum(jnp.dot(agg, sgw1a, precision=_HI) + rmean * sgw1b + sgb1_ref[...], 0.0)
    spre = jnp.dot(hsg, sgw2_ref[...], precision=_HI) + sgb2_ref[...]
    signew_ref[...] = jnp.maximum(spre, 0.0) + jnp.log1p(jnp.exp(-jnp.abs(spre)))


_SC_MESH = plsc.VectorSubcoreMesh(core_axis_name="c", subcore_axis_name="s")
_SC_PARAMS = pltpu.CompilerParams(needs_layout_passes=False)
_f32 = jnp.float32


def kernel(mu, sigma, edge_index, edge_dist, edge_conf, edge_angle, edge_depth_diff,
           msg_W1, msg_b1, msg_W2, msg_b2,
           mu_W1, mu_b1, mu_W2, mu_b2,
           sig_W1, sig_b1, sig_W2, sig_b2):
    src = edge_index[0]
    dst = edge_index[1]
    flat = src * N + dst
    d = edge_dist[:, 0]
    ef4 = jnp.concatenate([edge_dist, edge_conf, edge_angle, edge_depth_diff], axis=-1)

    p_nodes, ef_proj = pl.pallas_call(
        _proj_body,
        out_shape=(jax.ShapeDtypeStruct((N, H), _f32),
                   jax.ShapeDtypeStruct((E, H), _f32)),
    )(mu, sigma, ef4, msg_W1, msg_b1.reshape(1, H))

    adj_flat = pl.kernel(
        _adj_body,
        out_type=jax.ShapeDtypeStruct((N * N,), _f32),
        mesh=_SC_MESH,
        compiler_params=_SC_PARAMS,
        scratch_types=[
            pltpu.VMEM((E,), jnp.int32),
            pltpu.VMEM((E,), _f32),
            pltpu.VMEM((CELLS_W,), _f32),
        ],
    )(flat, d, jnp.full((CELLS_W,), -1.0, _f32))

    g_mat = pl.pallas_call(
        _twohop_body,
        out_shape=jax.ShapeDtypeStruct((N, N), _f32),
    )(adj_flat.reshape(N, N))

    srcr = src.reshape(NW, NCHUNK, CHUNK)
    dstr = dst.reshape(NW, NCHUNK, CHUNK)
    flatr = flat.reshape(NW, NCHUNK, CHUNK)
    dr = d.reshape(NW, NCHUNK, CHUNK)
    efr = ef_proj.reshape(NW, NCHUNK, CHUNK, H)
    agg_p, wsum_p, rsum_p, deg_p, resid = pl.kernel(
        _edge_body,
        out_type=(jax.ShapeDtypeStruct((NC, N, H), _f32),
                  jax.ShapeDtypeStruct((NC, N), _f32),
                  jax.ShapeDtypeStruct((NC, N), _f32),
                  jax.ShapeDtypeStruct((NC, N), _f32),
                  jax.ShapeDtypeStruct((NW, NCHUNK, CHUNK), _f32)),
        mesh=_SC_MESH,
        compiler_params=_SC_PARAMS,
        scratch_types=(
            [pltpu.VMEM((CHUNK,), jnp.int32)] * 6
            + [pltpu.VMEM((CHUNK,), _f32)] * 8
            + [
                pltpu.VMEM((CHUNK,), _f32),
                pltpu.VMEM((ROWS_S,), _f32),
                pltpu.VMEM((CHUNK, H), _f32),
                pltpu.VMEM((CHUNK, H), _f32),
                pltpu.VMEM_SHARED((N, H), _f32),
                pltpu.VMEM_SHARED((N,), _f32),
                pltpu.VMEM_SHARED((N,), _f32),
                pltpu.VMEM_SHARED((N,), _f32),
                pltpu.SemaphoreType.DMA,
            ]
        ),
    )(srcr, dstr, flatr, dr, g_mat.reshape(N * N), p_nodes, efr)

    mu_new, sigma_new = pl.pallas_call(
        _final_body,
        out_shape=(jax.ShapeDtypeStruct((N, H), _f32),
                   jax.ShapeDtypeStruct((N, H), _f32)),
    )(agg_p, wsum_p.reshape(NC, N, 1), rsum_p.reshape(NC, N, 1),
      deg_p.reshape(NC, N, 1), mu,
      msg_W2, msg_b2.reshape(1, H), mu_W1, mu_b1.reshape(1, H),
      mu_W2, mu_b2.reshape(1, H), sig_W1, sig_b1.reshape(1, H),
      sig_W2, sig_b2.reshape(1, H))

    return (mu_new, sigma_new, resid.reshape(E, 1))


# speedup vs baseline: 5.0306x; 1.3447x over previous
import functools

import jax
import jax.numpy as jnp
from jax import lax
from jax.experimental import pallas as pl
from jax.experimental.pallas import tpu as pltpu
from jax.experimental.pallas import tpu_sc as plsc

N = 1024
E = 16384
H = 128
NC = 2
NS = 16
NW = NC * NS
ROWS_W = N // NW
CELLS_W = ROWS_W * N
EW = E // NW
CHUNK = 256
NCHUNK = EW // CHUNK
ROWS_S = N // NS

_HI = jax.lax.Precision.HIGHEST


def _proj_body(mu_ref, sig_ref, ef4_ref, w1_ref, b1_ref, p_ref, efp_ref):
    w1a = w1_ref[0:H, :]
    w1b = w1_ref[H:2 * H, :]
    w1c = w1_ref[2 * H:, :]
    p_ref[...] = (jnp.dot(mu_ref[...], w1a, precision=_HI)
                  + jnp.dot(sig_ref[...], w1b, precision=_HI)
                  + b1_ref[...])
    efp_ref[...] = jnp.dot(ef4_ref[...], w1c, precision=_HI)


def _adj_body(flat_hbm, dval_hbm, fill_hbm, adj_hbm, idx_v, val_v, tile_v):
    c = lax.axis_index("c")
    s = lax.axis_index("s")
    wid = c * NS + s
    base = wid * CELLS_W
    pltpu.sync_copy(fill_hbm, tile_v)
    pltpu.sync_copy(flat_hbm, idx_v)
    pltpu.sync_copy(dval_hbm, val_v)

    def body(g, carry):
        for u in range(4):
            sl = pl.ds(g * 64 + u * 16, 16)
            idx = idx_v[sl]
            val = val_v[sl]
            loc = idx - base
            msk = (loc >= 0) & (loc < CELLS_W)
            locc = jnp.clip(loc, 0, CELLS_W - 1)
            plsc.store_scatter(tile_v, [locc >> 10, locc & (N - 1)], val, mask=msk)
        return carry

    lax.fori_loop(0, E // 64, body, 0)
    pltpu.sync_copy(tile_v, adj_hbm.at[pl.ds(wid * ROWS_W, ROWS_W)])


def _twohop_body(adj_ref, g_ref):
    a = adj_ref[...]
    m = (a >= 0.0).astype(jnp.float32)
    dm = jnp.maximum(a, 0.0)
    pc = jnp.dot(m, m)
    s = jnp.dot(dm, m) + jnp.dot(m, dm)
    g = jnp.where(pc > 0.0, s / jnp.maximum(pc, 1.0), -1.0)
    g_ref[...] = jnp.reshape(g, (N * N,))


def _edge_body(srcr_hbm, dstr_hbm, flatr_hbm, dr_hbm, g_hbm, pn_hbm, efr_hbm,
               agg_hbm, wsum_hbm, rsum_hbm, deg_hbm, resid_hbm,
               src_a, src_b, dst_a, dst_b, flat_a, flat_b, d_a, d_b,
               g_a, g_b, w_a, w_b, r_a, r_b, one_v, zv_v,
               prow_v, efrow_v,
               agg_s, wsum_s, rsum_s, deg_s, sem):
    src_c = (src_a, src_b)
    dst_c = (dst_a, dst_b)
    flat_c = (flat_a, flat_b)
    d_c = (d_a, d_b)
    g_c = (g_a, g_b)
    w_c = (w_a, w_b)
    r_c = (r_a, r_b)
    c = lax.axis_index("c")
    s = lax.axis_index("s")
    wid = c * NS + s

    zsl = pl.ds(s * ROWS_S, ROWS_S)
    zero16 = jnp.zeros((16,), _f32)

    def zrow_body(i, carry):
        for j in range(H // 16):
            efrow_v[i, pl.ds(j * 16, 16)] = zero16
        return carry

    lax.fori_loop(0, ROWS_S, zrow_body, 0)
    for g16 in range(ROWS_S // 16):
        zv_v[pl.ds(g16 * 16, 16)] = zero16
    pltpu.sync_copy(efrow_v.at[pl.ds(0, ROWS_S)], agg_s.at[zsl])
    pltpu.sync_copy(zv_v, wsum_s.at[zsl])
    pltpu.sync_copy(zv_v, rsum_s.at[zsl])
    pltpu.sync_copy(zv_v, deg_s.at[zsl])

    for ci in range(NCHUNK):
        pltpu.sync_copy(srcr_hbm.at[wid, ci], src_c[ci])
        pltpu.sync_copy(dstr_hbm.at[wid, ci], dst_c[ci])
        pltpu.sync_copy(flatr_hbm.at[wid, ci], flat_c[ci])
        pltpu.sync_copy(dr_hbm.at[wid, ci], d_c[ci])
    for g16 in range(16):
        one_v[pl.ds(g16 * 16, 16)] = jnp.full((16,), 1.0, jnp.float32)

    plsc.subcore_barrier()

    for ci in range(NCHUNK):
        pltpu.async_copy(g_hbm.at[flat_c[ci]], g_c[ci], sem).wait()
        for g16 in range(CHUNK // 16):
            sl = pl.ds(g16 * 16, 16)
            gg = g_c[ci][sl]
            dd = d_c[ci][sl]
            rr = jnp.where(gg >= 0.0, jnp.abs(dd - gg), 0.0)
            r_c[ci][sl] = rr
            w_c[ci][sl] = jnp.exp(-rr)
        pltpu.async_copy(pn_hbm.at[src_c[ci]], prow_v, sem).wait()
        pltpu.sync_copy(efr_hbm.at[wid, ci], efrow_v)
        w_ref = w_c[ci]

        def ebody(e, carry):
            wspl = plsc.load_gather(w_ref, [jnp.full((16,), e, jnp.int32)])
            for j in range(H // 16):
                sl = pl.ds(j * 16, 16)
                hv = jnp.maximum(prow_v[e, sl] + efrow_v[e, sl], 0.0) * wspl
                efrow_v[e, sl] = hv
            return carry

        lax.fori_loop(0, CHUNK, ebody, 0)

        pltpu.sync_copy(efrow_v, agg_s.at[dst_c[ci]], add=True)
        pltpu.sync_copy(w_c[ci], wsum_s.at[dst_c[ci]], add=True)
        pltpu.sync_copy(r_c[ci], rsum_s.at[dst_c[ci]], add=True)
        pltpu.sync_copy(one_v, deg_s.at[dst_c[ci]], add=True)

        pltpu.sync_copy(r_c[ci], resid_hbm.at[wid, ci])

    plsc.subcore_barrier()

    pltpu.sync_copy(agg_s.at[zsl], efrow_v.at[pl.ds(0, ROWS_S)])
    pltpu.sync_copy(efrow_v.at[pl.ds(0, ROWS_S)], agg_hbm.at[c, zsl])
    pltpu.sync_copy(wsum_s.at[zsl], zv_v)
    pltpu.sync_copy(zv_v, wsum_hbm.at[c, zsl])
    pltpu.sync_copy(rsum_s.at[zsl], zv_v)
    pltpu.sync_copy(zv_v, rsum_hbm.at[c, zsl])
    pltpu.sync_copy(deg_s.at[zsl], zv_v)
    pltpu.sync_copy(zv_v, deg_hbm.at[c, zsl])


def _final_body(agg_ref, ws_ref, rs_ref, dg_ref, mu_ref,
                w2_ref, b2_ref, muw1_ref, mub1_ref, muw2_ref, mub2_ref,
                sgw1_ref, sgb1_ref, sgw2_ref, sgb2_ref,
                munew_ref, signew_ref):
    a = agg_ref[0] + agg_ref[1]
    ws = ws_ref[0] + ws_ref[1]
    rs = rs_ref[0] + rs_ref[1]
    dg = dg_ref[0] + dg_ref[1]
    agg = ((jnp.dot(a, w2_ref[...], precision=_HI) + ws * b2_ref[...])
           / jnp.maximum(ws, 1e-8))
    hmu = jnp.maximum(jnp.dot(agg, muw1_ref[...], precision=_HI) + mub1_ref[...], 0.0)
    munew_ref[...] = mu_ref[...] + jnp.dot(hmu, muw2_ref[...], precision=_HI) + mub2_ref[...]
    rmean = rs / jnp.maximum(dg, 1.0)
    sgw1a = sgw1_ref[0:H, :]
    sgw1b = sgw1_ref[H:H + 1, :]
    hsg = jnp.maximum(jnp.dot(agg, sgw1a, precision=_HI) + rmean * sgw1b + sgb1_ref[...], 0.0)
    spre = jnp.dot(hsg, sgw2_ref[...], precision=_HI) + sgb2_ref[...]
    signew_ref[...] = jnp.maximum(spre, 0.0) + jnp.log1p(jnp.exp(-jnp.abs(spre)))


_SC_MESH = plsc.VectorSubcoreMesh(core_axis_name="c", subcore_axis_name="s")
_SC_PARAMS = pltpu.CompilerParams(needs_layout_passes=False)
_f32 = jnp.float32


def kernel(mu, sigma, edge_index, edge_dist, edge_conf, edge_angle, edge_depth_diff,
           msg_W1, msg_b1, msg_W2, msg_b2,
           mu_W1, mu_b1, mu_W2, mu_b2,
           sig_W1, sig_b1, sig_W2, sig_b2):
    src = edge_index[0]
    dst = edge_index[1]
    flat = src * N + dst
    d = edge_dist[:, 0]
    ef4 = jnp.concatenate([edge_dist, edge_conf, edge_angle, edge_depth_diff], axis=-1)

    p_nodes, ef_proj = pl.pallas_call(
        _proj_body,
        out_shape=(jax.ShapeDtypeStruct((N, H), _f32),
                   jax.ShapeDtypeStruct((E, H), _f32)),
    )(mu, sigma, ef4, msg_W1, msg_b1.reshape(1, H))

    adj = pl.kernel(
        _adj_body,
        out_type=jax.ShapeDtypeStruct((N, N), _f32),
        mesh=_SC_MESH,
        compiler_params=_SC_PARAMS,
        scratch_types=[
            pltpu.VMEM((E,), jnp.int32),
            pltpu.VMEM((E,), _f32),
            pltpu.VMEM((ROWS_W, N), _f32),
        ],
    )(flat, d, jnp.full((ROWS_W, N), -1.0, _f32))

    g_flat = pl.pallas_call(
        _twohop_body,
        out_shape=jax.ShapeDtypeStruct((N * N,), _f32),
    )(adj)

    srcr = src.reshape(NW, NCHUNK, CHUNK)
    dstr = dst.reshape(NW, NCHUNK, CHUNK)
    flatr = flat.reshape(NW, NCHUNK, CHUNK)
    dr = d.reshape(NW, NCHUNK, CHUNK)
    efr = ef_proj.reshape(NW, NCHUNK, CHUNK, H)
    agg_p, wsum_p, rsum_p, deg_p, resid = pl.kernel(
        _edge_body,
        out_type=(jax.ShapeDtypeStruct((NC, N, H), _f32),
                  jax.ShapeDtypeStruct((NC, N), _f32),
                  jax.ShapeDtypeStruct((NC, N), _f32),
                  jax.ShapeDtypeStruct((NC, N), _f32),
                  jax.ShapeDtypeStruct((NW, NCHUNK, CHUNK), _f32)),
        mesh=_SC_MESH,
        compiler_params=_SC_PARAMS,
        scratch_types=(
            [pltpu.VMEM((CHUNK,), jnp.int32)] * 6
            + [pltpu.VMEM((CHUNK,), _f32)] * 8
            + [
                pltpu.VMEM((CHUNK,), _f32),
                pltpu.VMEM((ROWS_S,), _f32),
                pltpu.VMEM((CHUNK, H), _f32),
                pltpu.VMEM((CHUNK, H), _f32),
                pltpu.VMEM_SHARED((N, H), _f32),
                pltpu.VMEM_SHARED((N,), _f32),
                pltpu.VMEM_SHARED((N,), _f32),
                pltpu.VMEM_SHARED((N,), _f32),
                pltpu.SemaphoreType.DMA,
            ]
        ),
    )(srcr, dstr, flatr, dr, g_flat, p_nodes, efr)

    mu_new, sigma_new = pl.pallas_call(
        _final_body,
        out_shape=(jax.ShapeDtypeStruct((N, H), _f32),
                   jax.ShapeDtypeStruct((N, H), _f32)),
    )(agg_p, wsum_p.reshape(NC, N, 1), rsum_p.reshape(NC, N, 1),
      deg_p.reshape(NC, N, 1), mu,
      msg_W2, msg_b2.reshape(1, H), mu_W1, mu_b1.reshape(1, H),
      mu_W2, mu_b2.reshape(1, H), sig_W1, sig_b1.reshape(1, H),
      sig_W2, sig_b2.reshape(1, H))

    return (mu_new, sigma_new, resid.reshape(E, 1))


# speedup vs baseline: 7.0720x; 1.4058x over previous
import functools

import jax
import jax.numpy as jnp
from jax import lax
from jax.experimental import pallas as pl
from jax.experimental.pallas import tpu as pltpu
from jax.experimental.pallas import tpu_sc as plsc

N = 1024
E = 16384
H = 128
NC = 2
NS = 16
NW = NC * NS
ROWS_W = N // NW
CELLS_W = ROWS_W * N
EW = E // NW
CHUNK = 128
NCHUNK = EW // CHUNK
ROWS_S = N // NS


def _proj_body(mu_ref, sig_ref, ef4_ref, w1_ref, b1_ref, p_ref, efp_ref):
    w1a = w1_ref[0:H, :]
    w1b = w1_ref[H:2 * H, :]
    p_ref[...] = jnp.dot(mu_ref[...], w1a) + jnp.dot(sig_ref[...], w1b) + b1_ref[...]
    ef4 = ef4_ref[...]
    efp = (ef4[:, 0:1] * w1_ref[2 * H:2 * H + 1, :]
           + ef4[:, 1:2] * w1_ref[2 * H + 1:2 * H + 2, :]
           + ef4[:, 2:3] * w1_ref[2 * H + 2:2 * H + 3, :]
           + ef4[:, 3:4] * w1_ref[2 * H + 3:2 * H + 4, :])
    efp_ref[...] = jnp.reshape(efp, (NW, NCHUNK, CHUNK, H))


def _adj_body(flat_hbm, dval_hbm, fill_hbm, adj_hbm, idx_v, val_v, tile_v, sem):
    c = lax.axis_index("c")
    s = lax.axis_index("s")
    wid = c * NS + s
    base = wid * CELLS_W
    d1 = pltpu.async_copy(fill_hbm, tile_v, sem)
    d2 = pltpu.async_copy(flat_hbm, idx_v, sem)
    d3 = pltpu.async_copy(dval_hbm, val_v, sem)
    d1.wait()
    d2.wait()
    d3.wait()

    def body(g, carry):
        for u in range(4):
            sl = pl.ds(g * 64 + u * 16, 16)
            idx = idx_v[sl]
            val = val_v[sl]
            loc = idx - base
            msk = (loc >= 0) & (loc < CELLS_W)
            locc = jnp.clip(loc, 0, CELLS_W - 1)
            plsc.store_scatter(tile_v, [locc >> 10, locc & (N - 1)], val, mask=msk)
        return carry

    lax.fori_loop(0, E // 64, body, 0)
    pltpu.sync_copy(tile_v, adj_hbm.at[pl.ds(wid * ROWS_W, ROWS_W)])


def _twohop_body(adj_ref, g_ref):
    a = adj_ref[...]
    m = (a >= 0.0).astype(jnp.float32)
    dm = jnp.maximum(a, 0.0)
    pc = jnp.dot(m, m)
    s = jnp.dot(dm, m) + jnp.dot(m, dm)
    g = jnp.where(pc > 0.0, s / jnp.maximum(pc, 1.0), -1.0)
    g_ref[...] = jnp.reshape(g, (N * N,))


def _edge_body(srcr_hbm, dstr_hbm, dstw_hbm, flatw_hbm, dw_hbm, g_hbm, pn_hbm,
               efr_hbm,
               agg_hbm, wsum_hbm, rsum_hbm, deg_hbm, resid_hbm,
               src_0, src_1, src_2, src_3, dst_0, dst_1, dst_2, dst_3,
               dstw_v, flat_v, d_v, g_v, w_v, r_v, one_v, zv_v,
               prow_a, prow_b, ef_0, ef_1, ef_2, ef_3,
               agg_s, wsum_s, rsum_s, deg_s,
               sem_in, sem_a, sem_b, sem_g, sem_s):
    c = lax.axis_index("c")
    s = lax.axis_index("s")
    wid = c * NS + s
    src_c = (src_0, src_1, src_2, src_3)
    dst_c = (dst_0, dst_1, dst_2, dst_3)
    ef_c = (ef_0, ef_1, ef_2, ef_3)
    prow_c = (prow_a, prow_b)
    sem_c = (sem_a, sem_b)

    in_descs = []
    for ci in range(NCHUNK):
        in_descs.append(pltpu.async_copy(srcr_hbm.at[wid, ci], src_c[ci], sem_in))
        in_descs.append(pltpu.async_copy(dstr_hbm.at[wid, ci], dst_c[ci], sem_in))
    in_descs.append(pltpu.async_copy(dstw_hbm.at[wid], dstw_v, sem_in))
    in_descs.append(pltpu.async_copy(flatw_hbm.at[wid], flat_v, sem_in))
    in_descs.append(pltpu.async_copy(dw_hbm.at[wid], d_v, sem_in))

    zsl = pl.ds(s * ROWS_S, ROWS_S)
    zero16 = jnp.zeros((16,), _f32)

    def zrow_body(i, carry):
        for j in range(H // 16):
            ef_0[i, pl.ds(j * 16, 16)] = zero16
        return carry

    lax.fori_loop(0, ROWS_S, zrow_body, 0)
    for g16 in range(ROWS_S // 16):
        zv_v[pl.ds(g16 * 16, 16)] = zero16
    for g16 in range(EW // 16):
        one_v[pl.ds(g16 * 16, 16)] = jnp.full((16,), 1.0, _f32)
    pltpu.sync_copy(ef_0.at[pl.ds(0, ROWS_S)], agg_s.at[zsl])
    pltpu.sync_copy(zv_v, wsum_s.at[zsl])
    pltpu.sync_copy(zv_v, rsum_s.at[zsl])
    pltpu.sync_copy(zv_v, deg_s.at[zsl])

    plsc.subcore_barrier()

    for dd in in_descs:
        dd.wait()

    gd = pltpu.async_copy(g_hbm.at[flat_v], g_v, sem_g)

    def fire(ci):
        return (pltpu.async_copy(pn_hbm.at[src_c[ci]], prow_c[ci % 2], sem_c[ci % 2]),
                pltpu.async_copy(efr_hbm.at[wid, ci], ef_c[ci], sem_c[ci % 2]))

    row_descs = {0: fire(0), 1: fire(1)}

    gd.wait()
    for g16 in range(EW // 16):
        sl = pl.ds(g16 * 16, 16)
        gg = g_v[sl]
        dd = d_v[sl]
        rr = jnp.where(gg >= 0.0, jnp.abs(dd - gg), 0.0)
        r_v[sl] = rr
        w_v[sl] = jnp.exp(-rr)

    sc_descs = [
        pltpu.async_copy(w_v, wsum_s.at[dstw_v], sem_s, add=True),
        pltpu.async_copy(r_v, rsum_s.at[dstw_v], sem_s, add=True),
        pltpu.async_copy(one_v, deg_s.at[dstw_v], sem_s, add=True),
    ]
    pltpu.sync_copy(r_v, resid_hbm.at[wid])

    for ci in range(NCHUNK):
        da, db = row_descs[ci]
        da.wait()
        db.wait()
        pw = prow_c[ci % 2]
        eb = ef_c[ci]
        cbase = ci * CHUNK

        def ebody(e):
            wspl = plsc.load_gather(w_v, [jnp.full((16,), cbase + e, jnp.int32)])
            for j in range(H // 16):
                sl = pl.ds(j * 16, 16)
                eb[e, sl] = jnp.maximum(pw[e, sl] + eb[e, sl], 0.0) * wspl

        plsc.parallel_loop(0, CHUNK, 1, unroll=2)(ebody)

        sc_descs.append(pltpu.async_copy(eb, agg_s.at[dst_c[ci]], sem_s, add=True))
        if ci + 2 < NCHUNK:
            row_descs[ci + 2] = fire(ci + 2)

    for dd in sc_descs:
        dd.wait()

    plsc.subcore_barrier()

    pltpu.sync_copy(agg_s.at[zsl], ef_0.at[pl.ds(0, ROWS_S)])
    pltpu.sync_copy(ef_0.at[pl.ds(0, ROWS_S)], agg_hbm.at[c, zsl])
    pltpu.sync_copy(wsum_s.at[zsl], zv_v)
    pltpu.sync_copy(zv_v, wsum_hbm.at[c, zsl])
    pltpu.sync_copy(rsum_s.at[zsl], zv_v)
    pltpu.sync_copy(zv_v, rsum_hbm.at[c, zsl])
    pltpu.sync_copy(deg_s.at[zsl], zv_v)
    pltpu.sync_copy(zv_v, deg_hbm.at[c, zsl])


def _final_body(agg_ref, ws_ref, rs_ref, dg_ref, mu_ref,
                w2_ref, b2_ref, muw1_ref, mub1_ref, muw2_ref, mub2_ref,
                sgw1_ref, sgb1_ref, sgw2_ref, sgb2_ref,
                munew_ref, signew_ref):
    a = agg_ref[0] + agg_ref[1]
    ws = ws_ref[0] + ws_ref[1]
    rs = rs_ref[0] + rs_ref[1]
    dg = dg_ref[0] + dg_ref[1]
    agg = ((jnp.dot(a, w2_ref[...]) + ws * b2_ref[...])
           / jnp.maximum(ws, 1e-8))
    hmu = jnp.maximum(jnp.dot(agg, muw1_ref[...]) + mub1_ref[...], 0.0)
    munew_ref[...] = mu_ref[...] + jnp.dot(hmu, muw2_ref[...]) + mub2_ref[...]
    rmean = rs / jnp.maximum(dg, 1.0)
    sgw1a = sgw1_ref[0:H, :]
    sgw1b = sgw1_ref[H:H + 1, :]
    hsg = jnp.maximum(jnp.dot(agg, sgw1a) + rmean * sgw1b + sgb1_ref[...], 0.0)
    spre = jnp.dot(hsg, sgw2_ref[...]) + sgb2_ref[...]
    signew_ref[...] = jnp.maximum(spre, 0.0) + jnp.log1p(jnp.exp(-jnp.abs(spre)))


_SC_MESH = plsc.VectorSubcoreMesh(core_axis_name="c", subcore_axis_name="s")
_SC_PARAMS = pltpu.CompilerParams(needs_layout_passes=False)
_f32 = jnp.float32


def kernel(mu, sigma, edge_index, edge_dist, edge_conf, edge_angle, edge_depth_diff,
           msg_W1, msg_b1, msg_W2, msg_b2,
           mu_W1, mu_b1, mu_W2, mu_b2,
           sig_W1, sig_b1, sig_W2, sig_b2):
    src = edge_index[0]
    dst = edge_index[1]
    flat = src * N + dst
    d = edge_dist[:, 0]
    ef4 = jnp.concatenate([edge_dist, edge_conf, edge_angle, edge_depth_diff], axis=-1)

    p_nodes, ef_proj = pl.pallas_call(
        _proj_body,
        out_shape=(jax.ShapeDtypeStruct((N, H), _f32),
                   jax.ShapeDtypeStruct((NW, NCHUNK, CHUNK, H), _f32)),
    )(mu, sigma, ef4, msg_W1, msg_b1.reshape(1, H))

    adj = pl.kernel(
        _adj_body,
        out_type=jax.ShapeDtypeStruct((N, N), _f32),
        mesh=_SC_MESH,
        compiler_params=_SC_PARAMS,
        scratch_types=[
            pltpu.VMEM((E,), jnp.int32),
            pltpu.VMEM((E,), _f32),
            pltpu.VMEM((ROWS_W, N), _f32),
            pltpu.SemaphoreType.DMA,
        ],
    )(flat, d, jnp.full((ROWS_W, N), -1.0, _f32))

    g_flat = pl.pallas_call(
        _twohop_body,
        out_shape=jax.ShapeDtypeStruct((N * N,), _f32),
    )(adj)

    srcr = src.reshape(NW, NCHUNK, CHUNK)
    dstr = dst.reshape(NW, NCHUNK, CHUNK)
    dstw = dst.reshape(NW, EW)
    flatw = flat.reshape(NW, EW)
    dw = d.reshape(NW, EW)
    agg_p, wsum_p, rsum_p, deg_p, resid = pl.kernel(
        _edge_body,
        out_type=(jax.ShapeDtypeStruct((NC, N, H), _f32),
                  jax.ShapeDtypeStruct((NC, N), _f32),
                  jax.ShapeDtypeStruct((NC, N), _f32),
                  jax.ShapeDtypeStruct((NC, N), _f32),
                  jax.ShapeDtypeStruct((NW, EW), _f32)),
        mesh=_SC_MESH,
        compiler_params=_SC_PARAMS,
        scratch_types=(
            [pltpu.VMEM((CHUNK,), jnp.int32)] * 8
            + [
                pltpu.VMEM((EW,), jnp.int32),
                pltpu.VMEM((EW,), jnp.int32),
                pltpu.VMEM((EW,), _f32),
                pltpu.VMEM((EW,), _f32),
                pltpu.VMEM((EW,), _f32),
                pltpu.VMEM((EW,), _f32),
                pltpu.VMEM((EW,), _f32),
                pltpu.VMEM((ROWS_S,), _f32),
                pltpu.VMEM((CHUNK, H), _f32),
                pltpu.VMEM((CHUNK, H), _f32),
                pltpu.VMEM((CHUNK, H), _f32),
                pltpu.VMEM((CHUNK, H), _f32),
                pltpu.VMEM((CHUNK, H), _f32),
                pltpu.VMEM((CHUNK, H), _f32),
                pltpu.VMEM_SHARED((N, H), _f32),
                pltpu.VMEM_SHARED((N,), _f32),
                pltpu.VMEM_SHARED((N,), _f32),
                pltpu.VMEM_SHARED((N,), _f32),
                pltpu.SemaphoreType.DMA,
                pltpu.SemaphoreType.DMA,
                pltpu.SemaphoreType.DMA,
                pltpu.SemaphoreType.DMA,
                pltpu.SemaphoreType.DMA,
            ]
        ),
    )(srcr, dstr, dstw, flatw, dw, g_flat, p_nodes, ef_proj)

    mu_new, sigma_new = pl.pallas_call(
        _final_body,
        out_shape=(jax.ShapeDtypeStruct((N, H), _f32),
                   jax.ShapeDtypeStruct((N, H), _f32)),
    )(agg_p, wsum_p.reshape(NC, N, 1), rsum_p.reshape(NC, N, 1),
      deg_p.reshape(NC, N, 1), mu,
      msg_W2, msg_b2.reshape(1, H), mu_W1, mu_b1.reshape(1, H),
      mu_W2, mu_b2.reshape(1, H), sig_W1, sig_b1.reshape(1, H),
      sig_W2, sig_b2.reshape(1, H))

    return (mu_new, sigma_new, resid.reshape(E, 1))


# speedup vs baseline: 7.7143x; 1.0908x over previous
import functools

import jax
import jax.numpy as jnp
from jax import lax
from jax.experimental import pallas as pl
from jax.experimental.pallas import tpu as pltpu
from jax.experimental.pallas import tpu_sc as plsc

N = 1024
E = 16384
H = 128
NC = 2
NS = 16
NW = NC * NS
ROWS_W = N // NW
CELLS_W = ROWS_W * N
EW = E // NW
CHUNK = 128
NCHUNK = EW // CHUNK
ROWS_S = N // NS


def _proj_body(mu_ref, sig_ref, ef4_ref, w1_ref, b1_ref, p_ref, efp_ref):
    w1a = w1_ref[0:H, :]
    w1b = w1_ref[H:2 * H, :]
    p_ref[...] = jnp.dot(mu_ref[...], w1a) + jnp.dot(sig_ref[...], w1b) + b1_ref[...]
    ef4 = ef4_ref[...]
    efp = (ef4[:, 0:1] * w1_ref[2 * H:2 * H + 1, :]
           + ef4[:, 1:2] * w1_ref[2 * H + 1:2 * H + 2, :]
           + ef4[:, 2:3] * w1_ref[2 * H + 2:2 * H + 3, :]
           + ef4[:, 3:4] * w1_ref[2 * H + 3:2 * H + 4, :])
    efp_ref[...] = jnp.reshape(efp, (NW, NCHUNK, CHUNK, H))


def _adj_body(flat_hbm, dval_hbm, fill_hbm, adj_hbm, idx_v, val_v, tile_v, sem):
    c = lax.axis_index("c")
    s = lax.axis_index("s")
    wid = c * NS + s
    d1 = pltpu.async_copy(fill_hbm, tile_v, sem)
    d2 = pltpu.async_copy(flat_hbm, idx_v, sem)
    d3 = pltpu.async_copy(dval_hbm, val_v, sem)
    d1.wait()
    d2.wait()
    d3.wait()

    row0 = wid * ROWS_W

    def body(g, carry):
        for u in range(8):
            sl = pl.ds(g * 128 + u * 16, 16)
            idx = idx_v[sl]
            val = val_v[sl]
            rloc = (idx >> 10) - row0
            msk = (rloc >= 0) & (rloc < ROWS_W)
            plsc.store_scatter(tile_v, [rloc, idx & (N - 1)], val, mask=msk)
        return carry

    lax.fori_loop(0, E // 128, body, 0)
    pltpu.sync_copy(tile_v, adj_hbm.at[pl.ds(wid * ROWS_W, ROWS_W)])


def _twohop_body(adj_ref, g_ref):
    a = adj_ref[...]
    m = (a >= 0.0).astype(jnp.float32)
    dm = jnp.maximum(a, 0.0)
    pc = jnp.dot(m, m)
    s = jnp.dot(dm, m) + jnp.dot(m, dm)
    g = jnp.where(pc > 0.0, s / jnp.maximum(pc, 1.0), -1.0)
    g_ref[...] = jnp.reshape(g, (N * N,))


def _edge_body(srcr_hbm, dstr_hbm, dstw_hbm, flatw_hbm, dw_hbm, g_hbm, pn_hbm,
               efr_hbm,
               agg_hbm, wsum_hbm, rsum_hbm, deg_hbm, resid_hbm,
               src_0, src_1, src_2, src_3, dst_0, dst_1, dst_2, dst_3,
               dstw_v, flat_v, d_v, g_v, w_v, r_v, one_v, zv_v,
               prow_a, prow_b, ef_0, ef_1, ef_2, ef_3,
               agg_s, wsum_s, rsum_s, deg_s,
               sem_in, sem_a, sem_b, sem_g, sem_s):
    c = lax.axis_index("c")
    s = lax.axis_index("s")
    wid = c * NS + s
    src_c = (src_0, src_1, src_2, src_3)
    dst_c = (dst_0, dst_1, dst_2, dst_3)
    ef_c = (ef_0, ef_1, ef_2, ef_3)
    prow_c = (prow_a, prow_b)
    sem_c = (sem_a, sem_b)

    in_descs = []
    for ci in range(NCHUNK):
        in_descs.append(pltpu.async_copy(srcr_hbm.at[wid, ci], src_c[ci], sem_in))
        in_descs.append(pltpu.async_copy(dstr_hbm.at[wid, ci], dst_c[ci], sem_in))
    in_descs.append(pltpu.async_copy(dstw_hbm.at[wid], dstw_v, sem_in))
    in_descs.append(pltpu.async_copy(flatw_hbm.at[wid], flat_v, sem_in))
    in_descs.append(pltpu.async_copy(dw_hbm.at[wid], d_v, sem_in))

    zsl = pl.ds(s * ROWS_S, ROWS_S)
    zero16 = jnp.zeros((16,), _f32)

    def zrow_body(i, carry):
        for j in range(H // 16):
            ef_0[i, pl.ds(j * 16, 16)] = zero16
        return carry

    lax.fori_loop(0, ROWS_S, zrow_body, 0)
    for g16 in range(ROWS_S // 16):
        zv_v[pl.ds(g16 * 16, 16)] = zero16
    for g16 in range(EW // 16):
        one_v[pl.ds(g16 * 16, 16)] = jnp.full((16,), 1.0, _f32)
    pltpu.sync_copy(ef_0.at[pl.ds(0, ROWS_S)], agg_s.at[zsl])
    pltpu.sync_copy(zv_v, wsum_s.at[zsl])
    pltpu.sync_copy(zv_v, rsum_s.at[zsl])
    pltpu.sync_copy(zv_v, deg_s.at[zsl])

    plsc.subcore_barrier()

    for dd in in_descs:
        dd.wait()

    gd = pltpu.async_copy(g_hbm.at[flat_v], g_v, sem_g)

    def fire(ci):
        return (pltpu.async_copy(pn_hbm.at[src_c[ci]], prow_c[ci % 2], sem_c[ci % 2]),
                pltpu.async_copy(efr_hbm.at[wid, ci], ef_c[ci], sem_c[ci % 2]))

    row_descs = {0: fire(0), 1: fire(1)}

    gd.wait()
    for g16 in range(EW // 16):
        sl = pl.ds(g16 * 16, 16)
        gg = g_v[sl]
        dd = d_v[sl]
        rr = jnp.where(gg >= 0.0, jnp.abs(dd - gg), 0.0)
        r_v[sl] = rr
        w_v[sl] = jnp.exp(-rr)

    sc_descs = [
        pltpu.async_copy(w_v, wsum_s.at[dstw_v], sem_s, add=True),
        pltpu.async_copy(r_v, rsum_s.at[dstw_v], sem_s, add=True),
        pltpu.async_copy(one_v, deg_s.at[dstw_v], sem_s, add=True),
    ]
    pltpu.sync_copy(r_v, resid_hbm.at[wid])

    for ci in range(NCHUNK):
        da, db = row_descs[ci]
        da.wait()
        db.wait()
        pw = prow_c[ci % 2]
        eb = ef_c[ci]
        cbase = ci * CHUNK

        def ebody(e):
            wspl = plsc.load_gather(w_v, [jnp.full((16,), cbase + e, jnp.int32)])
            for j in range(H // 16):
                sl = pl.ds(j * 16, 16)
                eb[e, sl] = jnp.maximum(pw[e, sl] + eb[e, sl], 0.0) * wspl

        plsc.parallel_loop(0, CHUNK, 1, unroll=4)(ebody)

        sc_descs.append(pltpu.async_copy(eb, agg_s.at[dst_c[ci]], sem_s, add=True))
        if ci + 2 < NCHUNK:
            row_descs[ci + 2] = fire(ci + 2)

    for dd in sc_descs:
        dd.wait()

    plsc.subcore_barrier()

    pltpu.sync_copy(agg_s.at[zsl], ef_0.at[pl.ds(0, ROWS_S)])
    pltpu.sync_copy(ef_0.at[pl.ds(0, ROWS_S)], agg_hbm.at[c, zsl])
    pltpu.sync_copy(wsum_s.at[zsl], zv_v)
    pltpu.sync_copy(zv_v, wsum_hbm.at[c, zsl])
    pltpu.sync_copy(rsum_s.at[zsl], zv_v)
    pltpu.sync_copy(zv_v, rsum_hbm.at[c, zsl])
    pltpu.sync_copy(deg_s.at[zsl], zv_v)
    pltpu.sync_copy(zv_v, deg_hbm.at[c, zsl])


def _final_body(agg_ref, ws_ref, rs_ref, dg_ref, mu_ref,
                w2_ref, b2_ref, muw1_ref, mub1_ref, muw2_ref, mub2_ref,
                sgw1_ref, sgb1_ref, sgw2_ref, sgb2_ref,
                munew_ref, signew_ref):
    a = agg_ref[0] + agg_ref[1]
    ws = jnp.reshape(ws_ref[0] + ws_ref[1], (N, 1))
    rs = jnp.reshape(rs_ref[0] + rs_ref[1], (N, 1))
    dg = jnp.reshape(dg_ref[0] + dg_ref[1], (N, 1))
    agg = ((jnp.dot(a, w2_ref[...]) + ws * b2_ref[...])
           / jnp.maximum(ws, 1e-8))
    hmu = jnp.maximum(jnp.dot(agg, muw1_ref[...]) + mub1_ref[...], 0.0)
    munew_ref[...] = mu_ref[...] + jnp.dot(hmu, muw2_ref[...]) + mub2_ref[...]
    rmean = rs / jnp.maximum(dg, 1.0)
    sgw1a = sgw1_ref[0:H, :]
    sgw1b = sgw1_ref[H:H + 1, :]
    hsg = jnp.maximum(jnp.dot(agg, sgw1a) + rmean * sgw1b + sgb1_ref[...], 0.0)
    spre = jnp.dot(hsg, sgw2_ref[...]) + sgb2_ref[...]
    signew_ref[...] = jnp.maximum(spre, 0.0) + jnp.log1p(jnp.exp(-jnp.abs(spre)))


_SC_MESH = plsc.VectorSubcoreMesh(core_axis_name="c", subcore_axis_name="s")
_SC_PARAMS = pltpu.CompilerParams(needs_layout_passes=False)
_f32 = jnp.float32


def kernel(mu, sigma, edge_index, edge_dist, edge_conf, edge_angle, edge_depth_diff,
           msg_W1, msg_b1, msg_W2, msg_b2,
           mu_W1, mu_b1, mu_W2, mu_b2,
           sig_W1, sig_b1, sig_W2, sig_b2):
    src = edge_index[0]
    dst = edge_index[1]
    flat = src * N + dst
    d = edge_dist[:, 0]
    ef4 = jnp.concatenate([edge_dist, edge_conf, edge_angle, edge_depth_diff], axis=-1)

    p_nodes, ef_proj = pl.pallas_call(
        _proj_body,
        out_shape=(jax.ShapeDtypeStruct((N, H), _f32),
                   jax.ShapeDtypeStruct((NW, NCHUNK, CHUNK, H), _f32)),
    )(mu, sigma, ef4, msg_W1, msg_b1.reshape(1, H))

    adj = pl.kernel(
        _adj_body,
        out_type=jax.ShapeDtypeStruct((N, N), _f32),
        mesh=_SC_MESH,
        compiler_params=_SC_PARAMS,
        scratch_types=[
            pltpu.VMEM((E,), jnp.int32),
            pltpu.VMEM((E,), _f32),
            pltpu.VMEM((ROWS_W, N), _f32),
            pltpu.SemaphoreType.DMA,
        ],
    )(flat, d, jnp.full((ROWS_W, N), -1.0, _f32))

    g_flat = pl.pallas_call(
        _twohop_body,
        out_shape=jax.ShapeDtypeStruct((N * N,), _f32),
    )(adj)

    srcr = src.reshape(NW, NCHUNK, CHUNK)
    dstr = dst.reshape(NW, NCHUNK, CHUNK)
    dstw = dst.reshape(NW, EW)
    flatw = flat.reshape(NW, EW)
    dw = d.reshape(NW, EW)
    agg_p, wsum_p, rsum_p, deg_p, resid = pl.kernel(
        _edge_body,
        out_type=(jax.ShapeDtypeStruct((NC, N, H), _f32),
                  jax.ShapeDtypeStruct((NC, N), _f32),
                  jax.ShapeDtypeStruct((NC, N), _f32),
                  jax.ShapeDtypeStruct((NC, N), _f32),
                  jax.ShapeDtypeStruct((NW, EW), _f32)),
        mesh=_SC_MESH,
        compiler_params=_SC_PARAMS,
        scratch_types=(
            [pltpu.VMEM((CHUNK,), jnp.int32)] * 8
            + [
                pltpu.VMEM((EW,), jnp.int32),
                pltpu.VMEM((EW,), jnp.int32),
                pltpu.VMEM((EW,), _f32),
                pltpu.VMEM((EW,), _f32),
                pltpu.VMEM((EW,), _f32),
                pltpu.VMEM((EW,), _f32),
                pltpu.VMEM((EW,), _f32),
                pltpu.VMEM((ROWS_S,), _f32),
                pltpu.VMEM((CHUNK, H), _f32),
                pltpu.VMEM((CHUNK, H), _f32),
                pltpu.VMEM((CHUNK, H), _f32),
                pltpu.VMEM((CHUNK, H), _f32),
                pltpu.VMEM((CHUNK, H), _f32),
                pltpu.VMEM((CHUNK, H), _f32),
                pltpu.VMEM_SHARED((N, H), _f32),
                pltpu.VMEM_SHARED((N,), _f32),
                pltpu.VMEM_SHARED((N,), _f32),
                pltpu.VMEM_SHARED((N,), _f32),
                pltpu.SemaphoreType.DMA,
                pltpu.SemaphoreType.DMA,
                pltpu.SemaphoreType.DMA,
                pltpu.SemaphoreType.DMA,
                pltpu.SemaphoreType.DMA,
            ]
        ),
    )(srcr, dstr, dstw, flatw, dw, g_flat, p_nodes, ef_proj)

    mu_new, sigma_new = pl.pallas_call(
        _final_body,
        out_shape=(jax.ShapeDtypeStruct((N, H), _f32),
                   jax.ShapeDtypeStruct((N, H), _f32)),
    )(agg_p, wsum_p, rsum_p, deg_p, mu,
      msg_W2, msg_b2.reshape(1, H), mu_W1, mu_b1.reshape(1, H),
      mu_W2, mu_b2.reshape(1, H), sig_W1, sig_b1.reshape(1, H),
      sig_W2, sig_b2.reshape(1, H))

    return (mu_new, sigma_new, resid.reshape(E, 1))
